# Initial kernel scaffold; baseline (speedup 1.0000x reference)
import jax
import jax.numpy as jnp
from jax.experimental import pallas as pl


def kernel(node_features, edge_features, edge_idx, params):
    raise NotImplementedError("write your pallas kernel here")


# speedup vs baseline: 6.3916x; 6.3916x over previous
import functools

import jax
import jax.numpy as jnp
from jax.experimental import pallas as pl

B = 16384
N = 5
N_EDGES = 20
HID = 64
N_LAYERS = 4
G = 128


def _silu(x):
    return x * jax.nn.sigmoid(x)


def _dot(a, b):
    return jnp.dot(a, b, preferred_element_type=jnp.float32)


def _gnn_kernel(x_ref, ef0_ref, ef1_ref, ei_ref, *wrefs, out_ref):
    ws = [w[:] for w in wrefs]
    it = iter(ws)
    W_emb, b_emb = next(it), next(it)
    layers = [tuple(next(it) for _ in range(11)) for _ in range(N_LAYERS)]
    W_d1, b_d1, W_d2, b_d2 = next(it), next(it), next(it), next(it)

    ei = ei_ref[:]
    src = ei[:, :N_EDGES]
    tgt = ei[:, N_EDGES:]

    eidx = src * N + tgt
    ef0t = ef0_ref[:]
    ef1t = ef1_ref[:]
    c0 = jnp.zeros((G, N_EDGES), jnp.float32)
    c1 = jnp.zeros((G, N_EDGES), jnp.float32)
    for k in range(N * N):
        mk = (eidx == k).astype(jnp.float32)
        c0 = c0 + mk * ef0t[:, k:k + 1]
        c1 = c1 + mk * ef1t[:, k:k + 1]

    srcm = [(src == n).astype(jnp.float32) for n in range(N)]
    tgtm = [(tgt == n).astype(jnp.float32) for n in range(N)]

    x2d = _dot(x_ref[:], W_emb) + b_emb

    for li in range(N_LAYERS):
        eW1s, eW1t, eW1e, eb1, eW2, eb2, nW1x, nW1a, nb1, nW2, nb2 = layers[li]
        x3 = x2d.reshape(G, N, HID)
        src_f = jnp.zeros((G, N_EDGES, HID), jnp.float32)
        tgt_f = jnp.zeros((G, N_EDGES, HID), jnp.float32)
        for n in range(N):
            xn = x3[:, n:n + 1, :]
            src_f = src_f + srcm[n][:, :, None] * xn
            tgt_f = tgt_f + tgtm[n][:, :, None] * xn
        ef_c = (c0[:, :, None] * eW1e[0:1, :][None, :, :]
                + c1[:, :, None] * eW1e[1:2, :][None, :, :])
        h = (_dot(src_f.reshape(G * N_EDGES, HID), eW1s)
             + _dot(tgt_f.reshape(G * N_EDGES, HID), eW1t)
             + ef_c.reshape(G * N_EDGES, HID)
             + eb1)
        e = _silu(h)
        e = _silu(_dot(e, eW2) + eb2)
        e3 = e.reshape(G, N_EDGES, HID)
        aggs = [jnp.sum(srcm[n][:, :, None] * e3, axis=1, keepdims=True)
                for n in range(N)]
        agg2d = jnp.concatenate(aggs, axis=1).reshape(G * N, HID)
        hn = _silu(_dot(x2d, nW1x) + _dot(agg2d, nW1a) + nb1)
        x2d = _dot(hn, nW2) + nb2

    d = _silu(_dot(x2d, W_d1) + b_d1)
    out_ref[:] = _dot(d, W_d2) + b_d2


def _body(x_ref, ef0_ref, ef1_ref, ei_ref, *rest):
    _gnn_kernel(x_ref, ef0_ref, ef1_ref, ei_ref, *rest[:-1], out_ref=rest[-1])


@jax.jit
def kernel(node_features, edge_features, edge_idx, params):
    b = node_features.shape[0]
    x_in = node_features.transpose(0, 1, 3, 2).reshape(b * N, 3 * 2)
    ef_flat = edge_features.reshape(b, N * N, 2)
    ef0_in = ef_flat[:, :, 0]
    ef1_in = ef_flat[:, :, 1]
    ei_in = edge_idx.astype(jnp.int32).reshape(b, 2 * N_EDGES)

    weights = [params['W_emb'], params['b_emb'].reshape(1, HID)]
    for i in range(N_LAYERS):
        p = params[f'layer_{i}']
        weights += [
            p['eW1'][:HID], p['eW1'][HID:2 * HID], p['eW1'][2 * HID:],
            p['eb1'].reshape(1, HID),
            p['eW2'], p['eb2'].reshape(1, HID),
            p['nW1'][:HID], p['nW1'][HID:],
            p['nb1'].reshape(1, HID),
            p['nW2'], p['nb2'].reshape(1, HID),
        ]
    weights += [params['W_d1'], params['b_d1'].reshape(1, HID),
                params['W_d2'], params['b_d2'].reshape(1, 3)]

    grid = (b // G,)
    data_specs = [
        pl.BlockSpec((G * N, 3 * 2), lambda i: (i, 0)),
        pl.BlockSpec((G, N * N), lambda i: (i, 0)),
        pl.BlockSpec((G, N * N), lambda i: (i, 0)),
        pl.BlockSpec((G, 2 * N_EDGES), lambda i: (i, 0)),
    ]
    w_specs = [pl.BlockSpec(w.shape, functools.partial(lambda nd, i: (0,) * nd,
                                                       w.ndim))
               for w in weights]
    out = pl.pallas_call(
        _body,
        grid=grid,
        in_specs=data_specs + w_specs,
        out_specs=pl.BlockSpec((G * N, 3), lambda i: (i, 0)),
        out_shape=jax.ShapeDtypeStruct((b * N, 3), jnp.float32),
    )(x_in, ef0_in, ef1_in, ei_in, *weights)
    return out.reshape(b, N, 3)
